# Initial kernel scaffold; baseline (speedup 1.0000x reference)
import jax
import jax.numpy as jnp
from jax.experimental import pallas as pl


def kernel(x, batch):
    raise NotImplementedError("write your pallas kernel here")


# speedup vs baseline: 5.9631x; 5.9631x over previous
import jax
import jax.numpy as jnp
from jax import lax
from jax.experimental import pallas as pl
from jax.experimental.pallas import tpu as pltpu
from jax.experimental.pallas import tpu_sc as plsc

N = 50000
D = 256
G = 512
KTOP = 3

NCORES = 2
NSUB = 16
NW = NCORES * NSUB
L = 16

CPT = 1568
RCH = 32
NCH = 50
KW = 512
KEYSB = 50176
KEYPAD = 52224

_NEG_INF = float("-inf")


def _iota16():
    return lax.iota(jnp.int32, L)


def _kernel_a_body(x_hbm, batch_hbm, part_hbm, keys_hbm,
                   xbuf, bbuf, kbuf, idxbuf, zbuf, acc):
    cid = lax.axis_index("c")
    sid = lax.axis_index("s")
    wid = cid * NSUB + sid
    base = wid * CPT
    bound = jnp.minimum(base + CPT, N)

    zero16 = jnp.zeros((L,), jnp.float32)

    def zrow(r, _):
        for c in range(D // L):
            zbuf[r, pl.ds(c * L, L)] = zero16
        return 0

    lax.fori_loop(0, RCH, zrow, 0)
    pltpu.sync_copy(zbuf, acc.at[pl.ds(sid * RCH, RCH)])

    @pl.when(sid == 0)
    def _():
        pltpu.sync_copy(zbuf.at[pl.ds(0, 1)], acc.at[pl.ds(G, 1)])

    plsc.subcore_barrier()

    col_last = jnp.full((L,), D - 1, jnp.int32)
    for k in range(NCH):
        s_u = base + k * RCH
        s = jnp.minimum(s_u, N - RCH)
        pltpu.sync_copy(x_hbm.at[pl.ds(s, RCH)], xbuf)
        pltpu.sync_copy(batch_hbm.at[pl.ds(s, RCH)], bbuf)
        for h in range(RCH // L):
            lanes = h * L + _iota16()
            rid = s + lanes
            valid = (rid >= s_u) & (rid < bound)
            bvals = bbuf[pl.ds(h * L, L)]
            idxbuf[pl.ds(h * L, L)] = jnp.where(valid, bvals, G)
            kbuf[pl.ds(h * L, L)] = plsc.load_gather(xbuf, [lanes, col_last])
        pltpu.sync_copy(xbuf, acc.at[idxbuf], add=True)
        pltpu.sync_copy(kbuf, keys_hbm.at[pl.ds(s, RCH)])

    plsc.subcore_barrier()
    pltpu.sync_copy(acc.at[pl.ds(sid * RCH, RCH)], xbuf)
    pltpu.sync_copy(xbuf, part_hbm.at[cid, pl.ds(sid * RCH, RCH)])


def _lower_bound(batchb, g):
    lo = jnp.zeros((L,), jnp.int32)
    hi = jnp.full((L,), N, jnp.int32)
    for _ in range(16):
        mid = jnp.minimum((lo + hi) >> 1, N - 1)
        bv = plsc.load_gather(batchb, [mid])
        pred = bv >= g
        hi = jnp.where(pred, mid, hi)
        lo = jnp.where(pred, lo, mid + 1)
    return lo


def _kernel_b_body(x_hbm, batch_hbm, part_hbm, keys_hbm,
                   mean_hbm, sums_hbm, topk_hbm,
                   batchb, keysb, rows, idx48, val48, invb, sem):
    cid = lax.axis_index("c")
    sid = lax.axis_index("s")
    wid = cid * NSUB + sid
    g0 = wid * L
    g = g0 + _iota16()

    pltpu.sync_copy(batch_hbm, batchb)
    start = _lower_bound(batchb, g)
    end = _lower_bound(batchb, g + 1)
    counts = end - start

    lo8 = jnp.min(start) & jnp.int32(~7)
    span = jnp.max(end) - lo8
    nkchunks = (span + (KW - 1)) >> 9

    def kstage(t, _):
        off = t * KW
        src = pl.multiple_of(lo8 + off, 8)
        pltpu.sync_copy(keys_hbm.at[pl.ds(src, KW)],
                        keysb.at[pl.ds(off, KW)])
        return 0

    lax.fori_loop(0, nkchunks, kstage, 0)

    start_l = start - lo8
    maxc = jnp.max(counts)

    def scan_body(j, carry):
        k1, k2, k3, i1, i2, i3 = carry
        m = j < counts
        kidx = jnp.where(m, start_l + j, 0)
        kj = plsc.load_gather(keysb, [kidx])
        kj = jnp.where(m, kj, _NEG_INF)
        gidx = start + j
        b1 = kj > k1
        b2 = kj > k2
        b3 = kj > k3
        nk3 = jnp.where(b3, jnp.where(b2, k2, kj), k3)
        ni3 = jnp.where(b3, jnp.where(b2, i2, gidx), i3)
        nk2 = jnp.where(b2, jnp.where(b1, k1, kj), k2)
        ni2 = jnp.where(b2, jnp.where(b1, i1, gidx), i2)
        nk1 = jnp.where(b1, kj, k1)
        ni1 = jnp.where(b1, gidx, i1)
        return nk1, nk2, nk3, ni1, ni2, ni3

    neg = jnp.full((L,), _NEG_INF, jnp.float32)
    zi = jnp.zeros((L,), jnp.int32)
    _, _, _, i1, i2, i3 = lax.fori_loop(
        0, maxc, scan_body, (neg, neg, neg, zi, zi, zi))

    one = jnp.ones((L,), jnp.float32)
    zerof = jnp.zeros((L,), jnp.float32)
    slot = _iota16() * KTOP
    for r, ir in enumerate((i1, i2, i3)):
        vr = counts > r
        plsc.store_scatter(idx48, [slot + r], jnp.where(vr, ir, 0))
        plsc.store_scatter(val48, [slot + r], jnp.where(vr, one, zerof))

    pltpu.async_copy(x_hbm.at[idx48], rows, sem).wait()

    def mask_row(r, _):
        mv = val48[pl.ds(r, L)][0]
        for c in range(D // L):
            sl = pl.ds(c * L, L)
            rows[r, sl] = rows[r, sl] * mv
        return 0

    lax.fori_loop(0, KTOP * L, mask_row, 0)
    pltpu.sync_copy(rows, topk_hbm.at[pl.ds(wid * KTOP * L, KTOP * L)])

    pltpu.sync_copy(part_hbm.at[0, pl.ds(g0, L)], rows.at[pl.ds(0, L)])
    pltpu.sync_copy(part_hbm.at[1, pl.ds(g0, L)], rows.at[pl.ds(L, L)])
    inv = 1.0 / jnp.maximum(counts.astype(jnp.float32), 1.0)
    invb[pl.ds(0, L)] = inv

    def sum_row(l, _):
        iv = invb[pl.ds(l, L)][0]
        for c in range(D // L):
            sl = pl.ds(c * L, L)
            sv = rows[l, sl] + rows[L + l, sl]
            rows[2 * L + l, sl] = sv
            rows[l, sl] = sv * iv
        return 0

    lax.fori_loop(0, L, sum_row, 0)
    pltpu.sync_copy(rows.at[pl.ds(2 * L, L)], sums_hbm.at[pl.ds(g0, L)])
    pltpu.sync_copy(rows.at[pl.ds(0, L)], mean_hbm.at[pl.ds(g0, L)])


def _make_kernels():
    mesh = plsc.VectorSubcoreMesh(core_axis_name="c", subcore_axis_name="s")
    params = pltpu.CompilerParams(use_tc_tiling_on_sc=False,
                                  needs_layout_passes=False)

    kernel_a = pl.kernel(
        _kernel_a_body,
        out_type=(
            jax.ShapeDtypeStruct((NCORES, G, D), jnp.float32),
            jax.ShapeDtypeStruct((KEYPAD,), jnp.float32),
        ),
        mesh=mesh,
        compiler_params=params,
        scratch_types=[
            pltpu.VMEM((RCH, D), jnp.float32),
            pltpu.VMEM((RCH,), jnp.int32),
            pltpu.VMEM((RCH,), jnp.float32),
            pltpu.VMEM((RCH,), jnp.int32),
            pltpu.VMEM((RCH, D), jnp.float32),
            pltpu.VMEM_SHARED((G + 1, D), jnp.float32),
        ],
    )

    kernel_b = pl.kernel(
        _kernel_b_body,
        out_type=(
            jax.ShapeDtypeStruct((G, D), jnp.float32),
            jax.ShapeDtypeStruct((G, D), jnp.float32),
            jax.ShapeDtypeStruct((G * KTOP, D), jnp.float32),
        ),
        mesh=mesh,
        compiler_params=params,
        scratch_types=[
            pltpu.VMEM((N,), jnp.int32),
            pltpu.VMEM((KEYSB,), jnp.float32),
            pltpu.VMEM((KTOP * L, D), jnp.float32),
            pltpu.VMEM((KTOP * L,), jnp.int32),
            pltpu.VMEM((KTOP * L + L,), jnp.float32),
            pltpu.VMEM((2 * L,), jnp.float32),
            pltpu.SemaphoreType.DMA,
        ],
    )
    return kernel_a, kernel_b


_KERNEL_A, _KERNEL_B = _make_kernels()


@jax.jit
def kernel(x, batch):
    batch = batch.astype(jnp.int32)
    partials, keys = _KERNEL_A(x, batch)
    mean, sums, topk = _KERNEL_B(x, batch, partials, keys)
    return jnp.concatenate([mean, sums, topk.reshape(G, KTOP * D)], axis=1)


# speedup vs baseline: 8.4301x; 1.4137x over previous
import jax
import jax.numpy as jnp
from jax import lax
from jax.experimental import pallas as pl
from jax.experimental.pallas import tpu as pltpu
from jax.experimental.pallas import tpu_sc as plsc

N = 50000
D = 256
G = 512
KTOP = 3

NCORES = 2
NSUB = 16
NW = NCORES * NSUB
L = 16

CPT = 1568
RCH = 128
NCH = 13
ZR = 32
KW = 512
KEYSB = 50176
KEYPAD = 52224

_NEG_INF = float("-inf")


def _iota16():
    return lax.iota(jnp.int32, L)


def _kernel_a_body(x_hbm, batch_hbm, part_hbm, keys_hbm,
                   xbuf0, xbuf1, bbuf0, bbuf1, kbuf0, kbuf1,
                   idxbuf0, idxbuf1, zbuf,
                   semx0, semx1, semb0, semb1, acc):
    cid = lax.axis_index("c")
    sid = lax.axis_index("s")
    wid = cid * NSUB + sid
    base = wid * CPT
    bound = jnp.minimum(base + CPT, N)

    zero16 = jnp.zeros((L,), jnp.float32)

    def zrow(r, _):
        for c in range(D // L):
            zbuf[r, pl.ds(c * L, L)] = zero16
        return 0

    lax.fori_loop(0, ZR, zrow, 0)
    pltpu.sync_copy(zbuf, acc.at[pl.ds(sid * ZR, ZR)])

    @pl.when(sid == 0)
    def _():
        pltpu.sync_copy(zbuf.at[pl.ds(0, 1)], acc.at[pl.ds(G, 1)])

    plsc.subcore_barrier()

    bufs = ((xbuf0, bbuf0, kbuf0, idxbuf0, semx0, semb0),
            (xbuf1, bbuf1, kbuf1, idxbuf1, semx1, semb1))

    def chunk_start(k):
        s_u = base + k * RCH
        return s_u, jnp.minimum(s_u, N - RCH)

    def issue(k, buf):
        xb, bb, _, _, sx, sb = buf
        _, s = chunk_start(k)
        hx = pltpu.async_copy(x_hbm.at[pl.ds(s, RCH)], xb, sx)
        hb = pltpu.async_copy(batch_hbm.at[pl.ds(s, RCH)], bb, sb)
        return hx, hb

    col_last = jnp.full((L,), D - 1, jnp.int32)
    pending = issue(0, bufs[0])
    for k in range(NCH):
        xb, bb, kb, ib, _, _ = bufs[k % 2]
        pending[0].wait()
        pending[1].wait()
        if k + 1 < NCH:
            pending = issue(k + 1, bufs[(k + 1) % 2])
        s_u, s = chunk_start(k)
        for h in range(RCH // L):
            lanes = h * L + _iota16()
            rid = s + lanes
            valid = (rid >= s_u) & (rid < bound)
            bvals = bb[pl.ds(h * L, L)]
            ib[pl.ds(h * L, L)] = jnp.where(valid, bvals, G)
            kb[pl.ds(h * L, L)] = plsc.load_gather(xb, [lanes, col_last])
        pltpu.sync_copy(xb, acc.at[ib], add=True)
        pltpu.sync_copy(kb, keys_hbm.at[pl.ds(s, RCH)])

    plsc.subcore_barrier()
    pltpu.sync_copy(acc.at[pl.ds(sid * ZR, ZR)], zbuf)
    pltpu.sync_copy(zbuf, part_hbm.at[cid, pl.ds(sid * ZR, ZR)])


def _lower_bound(batchb, g):
    lo = jnp.zeros((L,), jnp.int32)
    hi = jnp.full((L,), N, jnp.int32)
    for _ in range(16):
        mid = jnp.minimum((lo + hi) >> 1, N - 1)
        bv = plsc.load_gather(batchb, [mid])
        pred = bv >= g
        hi = jnp.where(pred, mid, hi)
        lo = jnp.where(pred, lo, mid + 1)
    return lo


def _kernel_b_body(x_hbm, batch_hbm, part_hbm, keys_hbm,
                   mean_hbm, sums_hbm, topk_hbm,
                   batchb, keysb, rows, idx48, val48, invb, sem):
    cid = lax.axis_index("c")
    sid = lax.axis_index("s")
    wid = cid * NSUB + sid
    g0 = wid * L
    g = g0 + _iota16()

    pltpu.sync_copy(batch_hbm, batchb)
    start = _lower_bound(batchb, g)
    end = _lower_bound(batchb, g + 1)
    counts = end - start

    lo8 = jnp.min(start) & jnp.int32(~7)
    span = jnp.max(end) - lo8
    nkchunks = (span + (KW - 1)) >> 9

    def kstage(t, _):
        off = t * KW
        src = pl.multiple_of(lo8 + off, 8)
        pltpu.sync_copy(keys_hbm.at[pl.ds(src, KW)],
                        keysb.at[pl.ds(off, KW)])
        return 0

    lax.fori_loop(0, nkchunks, kstage, 0)

    start_l = start - lo8
    maxc = jnp.max(counts)

    UNR = 4

    def scan_body(t, carry):
        j0 = t * UNR
        keys_u = []
        for u in range(UNR):
            j = j0 + u
            m = j < counts
            kidx = jnp.where(m, start_l + j, 0)
            kj = plsc.load_gather(keysb, [kidx])
            keys_u.append(jnp.where(m, kj, _NEG_INF))
        k1, k2, k3, i1, i2, i3 = carry
        for u, kj in enumerate(keys_u):
            gidx = start + (j0 + u)
            b1 = kj > k1
            b2 = kj > k2
            b3 = kj > k3
            k3 = jnp.where(b3, jnp.where(b2, k2, kj), k3)
            i3 = jnp.where(b3, jnp.where(b2, i2, gidx), i3)
            k2 = jnp.where(b2, jnp.where(b1, k1, kj), k2)
            i2 = jnp.where(b2, jnp.where(b1, i1, gidx), i2)
            k1 = jnp.where(b1, kj, k1)
            i1 = jnp.where(b1, gidx, i1)
        return k1, k2, k3, i1, i2, i3

    neg = jnp.full((L,), _NEG_INF, jnp.float32)
    zi = jnp.zeros((L,), jnp.int32)
    _, _, _, i1, i2, i3 = lax.fori_loop(
        0, (maxc + (UNR - 1)) >> 2, scan_body, (neg, neg, neg, zi, zi, zi))

    one = jnp.ones((L,), jnp.float32)
    zerof = jnp.zeros((L,), jnp.float32)
    slot = _iota16() * KTOP
    for r, ir in enumerate((i1, i2, i3)):
        vr = counts > r
        plsc.store_scatter(idx48, [slot + r], jnp.where(vr, ir, 0))
        plsc.store_scatter(val48, [slot + r], jnp.where(vr, one, zerof))

    pltpu.async_copy(x_hbm.at[idx48], rows, sem).wait()

    def mask_row(r, _):
        mv = val48[pl.ds(r, L)][0]
        for c in range(D // L):
            sl = pl.ds(c * L, L)
            rows[r, sl] = rows[r, sl] * mv
        return 0

    lax.fori_loop(0, KTOP * L, mask_row, 0)
    pltpu.sync_copy(rows, topk_hbm.at[pl.ds(wid * KTOP * L, KTOP * L)])

    pltpu.sync_copy(part_hbm.at[0, pl.ds(g0, L)], rows.at[pl.ds(0, L)])
    pltpu.sync_copy(part_hbm.at[1, pl.ds(g0, L)], rows.at[pl.ds(L, L)])
    inv = 1.0 / jnp.maximum(counts.astype(jnp.float32), 1.0)
    invb[pl.ds(0, L)] = inv

    def sum_row(l, _):
        iv = invb[pl.ds(l, L)][0]
        for c in range(D // L):
            sl = pl.ds(c * L, L)
            sv = rows[l, sl] + rows[L + l, sl]
            rows[2 * L + l, sl] = sv
            rows[l, sl] = sv * iv
        return 0

    lax.fori_loop(0, L, sum_row, 0)
    pltpu.sync_copy(rows.at[pl.ds(2 * L, L)], sums_hbm.at[pl.ds(g0, L)])
    pltpu.sync_copy(rows.at[pl.ds(0, L)], mean_hbm.at[pl.ds(g0, L)])


_AROWS = 64


def _assemble_body(mean_ref, sums_ref, topk_ref, out_ref):
    out_ref[:, 0:D] = mean_ref[...]
    out_ref[:, D:2 * D] = sums_ref[...]
    out_ref[:, 2 * D:] = topk_ref[...]


_ASSEMBLE = pl.pallas_call(
    _assemble_body,
    grid=(G // _AROWS,),
    in_specs=[
        pl.BlockSpec((_AROWS, D), lambda i: (i, 0)),
        pl.BlockSpec((_AROWS, D), lambda i: (i, 0)),
        pl.BlockSpec((_AROWS, KTOP * D), lambda i: (i, 0)),
    ],
    out_specs=pl.BlockSpec((_AROWS, (KTOP + 2) * D), lambda i: (i, 0)),
    out_shape=jax.ShapeDtypeStruct((G, (KTOP + 2) * D), jnp.float32),
)


def _make_kernels():
    mesh = plsc.VectorSubcoreMesh(core_axis_name="c", subcore_axis_name="s")
    params = pltpu.CompilerParams(use_tc_tiling_on_sc=False,
                                  needs_layout_passes=False)

    kernel_a = pl.kernel(
        _kernel_a_body,
        out_type=(
            jax.ShapeDtypeStruct((NCORES, G, D), jnp.float32),
            jax.ShapeDtypeStruct((KEYPAD,), jnp.float32),
        ),
        mesh=mesh,
        compiler_params=params,
        scratch_types=[
            pltpu.VMEM((RCH, D), jnp.float32),
            pltpu.VMEM((RCH, D), jnp.float32),
            pltpu.VMEM((RCH,), jnp.int32),
            pltpu.VMEM((RCH,), jnp.int32),
            pltpu.VMEM((RCH,), jnp.float32),
            pltpu.VMEM((RCH,), jnp.float32),
            pltpu.VMEM((RCH,), jnp.int32),
            pltpu.VMEM((RCH,), jnp.int32),
            pltpu.VMEM((ZR, D), jnp.float32),
            pltpu.SemaphoreType.DMA,
            pltpu.SemaphoreType.DMA,
            pltpu.SemaphoreType.DMA,
            pltpu.SemaphoreType.DMA,
            pltpu.VMEM_SHARED((G + 1, D), jnp.float32),
        ],
    )

    kernel_b = pl.kernel(
        _kernel_b_body,
        out_type=(
            jax.ShapeDtypeStruct((G, D), jnp.float32),
            jax.ShapeDtypeStruct((G, D), jnp.float32),
            jax.ShapeDtypeStruct((G * KTOP, D), jnp.float32),
        ),
        mesh=mesh,
        compiler_params=params,
        scratch_types=[
            pltpu.VMEM((N,), jnp.int32),
            pltpu.VMEM((KEYSB,), jnp.float32),
            pltpu.VMEM((KTOP * L, D), jnp.float32),
            pltpu.VMEM((KTOP * L,), jnp.int32),
            pltpu.VMEM((KTOP * L + L,), jnp.float32),
            pltpu.VMEM((2 * L,), jnp.float32),
            pltpu.SemaphoreType.DMA,
        ],
    )
    return kernel_a, kernel_b


_KERNEL_A, _KERNEL_B = _make_kernels()


@jax.jit
def kernel(x, batch):
    batch = batch.astype(jnp.int32)
    partials, keys = _KERNEL_A(x, batch)
    mean, sums, topk = _KERNEL_B(x, batch, partials, keys)
    return _ASSEMBLE(mean, sums, topk.reshape(G, KTOP * D))
